# baseline (speedup 1.0000x reference)
import functools

import jax
import jax.numpy as jnp
from jax.experimental import pallas as pl
from jax.experimental.pallas import tpu as pltpu

_B = 32
_N = 1_000_000
_W = 8192

_K0 = 0
_K1 = 42

_ROT_A = (13, 15, 26, 6)
_ROT_B = (17, 29, 16, 24)


def _rotl(x, d):
    return (x << jnp.uint32(d)) | (x >> jnp.uint32(32 - d))


def _threefry_bits(x0, x1):
    ks0 = jnp.uint32(_K0)
    ks1 = jnp.uint32(_K1)
    ks2 = jnp.uint32(_K0 ^ _K1 ^ 0x1BD11BDA)
    ks = (ks0, ks1, ks2)
    rots = (_ROT_A, _ROT_B)
    x0 = x0 + ks0
    x1 = x1 + ks1
    for i in range(5):
        for r in rots[i % 2]:
            x0 = x0 + x1
            x1 = _rotl(x1, r)
            x1 = x0 ^ x1
        x0 = x0 + ks[(i + 1) % 3]
        x1 = x1 + ks[(i + 2) % 3] + jnp.uint32(i + 1)
    return x0 ^ x1


def _gumbel_block(cols_u32):
    bits = _threefry_bits(jnp.zeros_like(cols_u32), cols_u32)
    fbits = (bits >> jnp.uint32(9)) | jnp.uint32(0x3F800000)
    u01 = jax.lax.bitcast_convert_type(fbits, jnp.float32) - jnp.float32(1.0)
    minval = jnp.float32(1e-7)
    maxval = jnp.float32(1.0 - 1e-7)
    u = jnp.maximum(minval, u01 * (maxval - minval) + minval)
    return -jnp.log(-jnp.log(u))


def _sample_kernel(logits_ref, idx_ref, max_ref, *, n_blocks):
    j = pl.program_id(0)
    base = j * _W
    col = jax.lax.broadcasted_iota(jnp.int32, (_B, _W), 1) + base
    row = jax.lax.broadcasted_iota(jnp.int32, (_B, _W), 0)
    lin = (row * _N + col).astype(jnp.uint32)
    g = _gumbel_block(lin)
    x = logits_ref[...] + g
    valid = col < _N
    x = jnp.where(valid, x, -jnp.inf)
    m = jnp.max(x, axis=1, keepdims=True)
    big = jnp.int32(0x7FFFFFFF)
    bidx = jnp.min(jnp.where(x == m, col, big), axis=1, keepdims=True)

    @pl.when(j == 0)
    def _init():
        max_ref[...] = m
        idx_ref[...] = bidx

    @pl.when(j > 0)
    def _update():
        better = m > max_ref[...]
        max_ref[...] = jnp.where(better, m, max_ref[...])
        idx_ref[...] = jnp.where(better, bidx, idx_ref[...])


def kernel(logits):
    n_blocks = pl.cdiv(_N, _W)
    idx, _ = pl.pallas_call(
        functools.partial(_sample_kernel, n_blocks=n_blocks),
        grid=(n_blocks,),
        in_specs=[pl.BlockSpec((_B, _W), lambda j: (0, j))],
        out_specs=[
            pl.BlockSpec((_B, 1), lambda j: (0, 0)),
            pl.BlockSpec((_B, 1), lambda j: (0, 0)),
        ],
        out_shape=[
            jax.ShapeDtypeStruct((_B, 1), jnp.int32),
            jax.ShapeDtypeStruct((_B, 1), jnp.float32),
        ],
    )(logits)
    return idx.reshape(_B)


# speedup vs baseline: 1.2865x; 1.2865x over previous
import jax
import jax.numpy as jnp
from jax.experimental import pallas as pl

_B = 32
_N = 1_000_000
_W = 8192
_C = 256

_K0 = 0
_K1 = 42

_ROT_A = (13, 15, 26, 6)
_ROT_B = (17, 29, 16, 24)


def _rotl(x, d):
    return (x << jnp.uint32(d)) | (x >> jnp.uint32(32 - d))


def _threefry_bits(x0, x1):
    ks0 = jnp.uint32(_K0)
    ks1 = jnp.uint32(_K1)
    ks2 = jnp.uint32(_K0 ^ _K1 ^ 0x1BD11BDA)
    ks = (ks0, ks1, ks2)
    rots = (_ROT_A, _ROT_B)
    x0 = x0 + ks0
    x1 = x1 + ks1
    for i in range(5):
        for r in rots[i % 2]:
            x0 = x0 + x1
            x1 = _rotl(x1, r)
            x1 = x0 ^ x1
        x0 = x0 + ks[(i + 1) % 3]
        x1 = x1 + ks[(i + 2) % 3] + jnp.uint32(i + 1)
    return x0 ^ x1


def _gumbel(lin_u32):
    bits = _threefry_bits(jnp.zeros_like(lin_u32), lin_u32)
    fbits = (bits >> jnp.uint32(9)) | jnp.uint32(0x3F800000)
    u01 = jax.lax.bitcast_convert_type(fbits, jnp.float32) - jnp.float32(1.0)
    minval = jnp.float32(1e-7)
    maxval = jnp.float32(1.0 - 1e-7)
    u = jnp.maximum(minval, u01 * (maxval - minval) + minval)
    return -jnp.log(-jnp.log(u))


def _sample_kernel(logits_ref, idx_ref, max_ref):
    j = pl.program_id(0)
    base = j * _W
    col0 = jax.lax.broadcasted_iota(jnp.int32, (_B, _C), 1)
    rowoff = jax.lax.broadcasted_iota(jnp.int32, (_B, _C), 0) * _N

    def body(k, carry):
        runm, runc = carry
        cbase = base + k * _C
        col = col0 + cbase
        lin = (rowoff + col).astype(jnp.uint32)
        x = logits_ref[:, pl.ds(k * _C, _C)] + _gumbel(lin)
        x = jnp.where(col < _N, x, -jnp.inf)
        better = x > runm
        runm = jnp.where(better, x, runm)
        runc = jnp.where(better, col, runc)
        return runm, runc

    init = (jnp.full((_B, _C), -jnp.inf, jnp.float32),
            jnp.zeros((_B, _C), jnp.int32))
    runm, runc = jax.lax.fori_loop(0, _W // _C, body, init)

    m = jnp.max(runm, axis=1, keepdims=True)
    big = jnp.int32(0x7FFFFFFF)
    bidx = jnp.min(jnp.where(runm == m, runc, big), axis=1, keepdims=True)

    @pl.when(j == 0)
    def _init():
        max_ref[...] = m
        idx_ref[...] = bidx

    @pl.when(j > 0)
    def _update():
        better = m > max_ref[...]
        max_ref[...] = jnp.where(better, m, max_ref[...])
        idx_ref[...] = jnp.where(better, bidx, idx_ref[...])


def kernel(logits):
    n_blocks = pl.cdiv(_N, _W)
    idx, _ = pl.pallas_call(
        _sample_kernel,
        grid=(n_blocks,),
        in_specs=[pl.BlockSpec((_B, _W), lambda j: (0, j))],
        out_specs=[
            pl.BlockSpec((_B, 1), lambda j: (0, 0)),
            pl.BlockSpec((_B, 1), lambda j: (0, 0)),
        ],
        out_shape=[
            jax.ShapeDtypeStruct((_B, 1), jnp.int32),
            jax.ShapeDtypeStruct((_B, 1), jnp.float32),
        ],
    )(logits)
    return idx.reshape(_B)


# speedup vs baseline: 1.2924x; 1.0047x over previous
import jax
import jax.numpy as jnp
from jax.experimental import pallas as pl

_B = 32
_N = 1_000_000
_W = 8192
_C = 512

_K0 = 0
_K1 = 42

_ROT_A = (13, 15, 26, 6)
_ROT_B = (17, 29, 16, 24)


def _rotl(x, d):
    return (x << jnp.uint32(d)) | (x >> jnp.uint32(32 - d))


def _threefry_bits(x0, x1):
    ks0 = jnp.uint32(_K0)
    ks1 = jnp.uint32(_K1)
    ks2 = jnp.uint32(_K0 ^ _K1 ^ 0x1BD11BDA)
    ks = (ks0, ks1, ks2)
    rots = (_ROT_A, _ROT_B)
    x0 = x0 + ks0
    x1 = x1 + ks1
    for i in range(5):
        for r in rots[i % 2]:
            x0 = x0 + x1
            x1 = _rotl(x1, r)
            x1 = x0 ^ x1
        x0 = x0 + ks[(i + 1) % 3]
        x1 = x1 + ks[(i + 2) % 3] + jnp.uint32(i + 1)
    return x0 ^ x1


def _gumbel(lin_u32):
    bits = _threefry_bits(jnp.zeros_like(lin_u32), lin_u32)
    fbits = (bits >> jnp.uint32(9)) | jnp.uint32(0x3F800000)
    u01 = jax.lax.bitcast_convert_type(fbits, jnp.float32) - jnp.float32(1.0)
    minval = jnp.float32(1e-7)
    maxval = jnp.float32(1.0 - 1e-7)
    u = jnp.maximum(minval, u01 * (maxval - minval) + minval)
    return -jnp.log(-jnp.log(u))


def _sample_kernel(logits_ref, idx_ref, max_ref):
    j = pl.program_id(0)
    base = j * _W
    col0 = jax.lax.broadcasted_iota(jnp.int32, (_B, _C), 1)
    rowoff = jax.lax.broadcasted_iota(jnp.int32, (_B, _C), 0) * _N

    def body(k, carry):
        runm, runc = carry
        cbase = base + k * _C
        col = col0 + cbase
        lin = (rowoff + col).astype(jnp.uint32)
        x = logits_ref[:, pl.ds(k * _C, _C)] + _gumbel(lin)
        x = jnp.where(col < _N, x, -jnp.inf)
        better = x > runm
        runm = jnp.where(better, x, runm)
        runc = jnp.where(better, col, runc)
        return runm, runc

    init = (jnp.full((_B, _C), -jnp.inf, jnp.float32),
            jnp.zeros((_B, _C), jnp.int32))
    runm, runc = jax.lax.fori_loop(0, _W // _C, body, init)

    m = jnp.max(runm, axis=1, keepdims=True)
    big = jnp.int32(0x7FFFFFFF)
    bidx = jnp.min(jnp.where(runm == m, runc, big), axis=1, keepdims=True)

    @pl.when(j == 0)
    def _init():
        max_ref[...] = m
        idx_ref[...] = bidx

    @pl.when(j > 0)
    def _update():
        better = m > max_ref[...]
        max_ref[...] = jnp.where(better, m, max_ref[...])
        idx_ref[...] = jnp.where(better, bidx, idx_ref[...])


def kernel(logits):
    n_blocks = pl.cdiv(_N, _W)
    idx, _ = pl.pallas_call(
        _sample_kernel,
        grid=(n_blocks,),
        in_specs=[pl.BlockSpec((_B, _W), lambda j: (0, j))],
        out_specs=[
            pl.BlockSpec((_B, 1), lambda j: (0, 0)),
            pl.BlockSpec((_B, 1), lambda j: (0, 0)),
        ],
        out_shape=[
            jax.ShapeDtypeStruct((_B, 1), jnp.int32),
            jax.ShapeDtypeStruct((_B, 1), jnp.float32),
        ],
    )(logits)
    return idx.reshape(_B)


# speedup vs baseline: 1.3488x; 1.0436x over previous
import jax
import jax.numpy as jnp
from jax.experimental import pallas as pl
from jax.experimental.pallas import tpu as pltpu

_B = 32
_N = 1_000_000
_W = 32768
_C = 512

_K0 = 0
_K1 = 42

_ROT_A = (13, 15, 26, 6)
_ROT_B = (17, 29, 16, 24)


def _rotl(x, d):
    return (x << jnp.uint32(d)) | (x >> jnp.uint32(32 - d))


def _threefry_bits(x0, x1):
    ks0 = jnp.uint32(_K0)
    ks1 = jnp.uint32(_K1)
    ks2 = jnp.uint32(_K0 ^ _K1 ^ 0x1BD11BDA)
    ks = (ks0, ks1, ks2)
    rots = (_ROT_A, _ROT_B)
    x0 = x0 + ks0
    x1 = x1 + ks1
    for i in range(5):
        for r in rots[i % 2]:
            x0 = x0 + x1
            x1 = _rotl(x1, r)
            x1 = x0 ^ x1
        x0 = x0 + ks[(i + 1) % 3]
        x1 = x1 + ks[(i + 2) % 3] + jnp.uint32(i + 1)
    return x0 ^ x1


def _gumbel(lin_u32):
    bits = _threefry_bits(jnp.zeros_like(lin_u32), lin_u32)
    fbits = (bits >> jnp.uint32(9)) | jnp.uint32(0x3F800000)
    u01 = jax.lax.bitcast_convert_type(fbits, jnp.float32) - jnp.float32(1.0)
    minval = jnp.float32(1e-7)
    maxval = jnp.float32(1.0 - 1e-7)
    u = jnp.maximum(minval, u01 * (maxval - minval) + minval)
    return -jnp.log(-jnp.log(u))


def _sample_kernel(logits_ref, idx_ref, max_ref, runm_ref, runc_ref):
    j = pl.program_id(0)
    n_steps = pl.num_programs(0)
    base = j * _W
    col0 = jax.lax.broadcasted_iota(jnp.int32, (_B, _C), 1)
    rowoff = jax.lax.broadcasted_iota(jnp.int32, (_B, _C), 0) * _N

    def body(k, carry):
        runm, runc = carry
        cbase = base + k * _C
        col = col0 + cbase
        lin = (rowoff + col).astype(jnp.uint32)
        x = logits_ref[:, pl.ds(k * _C, _C)] + _gumbel(lin)
        x = jnp.where(col < _N, x, -jnp.inf)
        better = x > runm
        runm = jnp.where(better, x, runm)
        runc = jnp.where(better, col, runc)
        return runm, runc

    @pl.when(j == 0)
    def _init():
        runm_ref[...] = jnp.full((_B, _C), -jnp.inf, jnp.float32)
        runc_ref[...] = jnp.zeros((_B, _C), jnp.int32)

    init = (runm_ref[...], runc_ref[...])
    runm, runc = jax.lax.fori_loop(0, _W // _C, body, init)
    runm_ref[...] = runm
    runc_ref[...] = runc

    @pl.when(j == n_steps - 1)
    def _finish():
        m = jnp.max(runm, axis=1, keepdims=True)
        big = jnp.int32(0x7FFFFFFF)
        bidx = jnp.min(jnp.where(runm == m, runc, big), axis=1, keepdims=True)
        max_ref[...] = m
        idx_ref[...] = bidx


def kernel(logits):
    n_blocks = pl.cdiv(_N, _W)
    idx, _ = pl.pallas_call(
        _sample_kernel,
        grid=(n_blocks,),
        in_specs=[pl.BlockSpec((_B, _W), lambda j: (0, j))],
        out_specs=[
            pl.BlockSpec((_B, 1), lambda j: (0, 0)),
            pl.BlockSpec((_B, 1), lambda j: (0, 0)),
        ],
        out_shape=[
            jax.ShapeDtypeStruct((_B, 1), jnp.int32),
            jax.ShapeDtypeStruct((_B, 1), jnp.float32),
        ],
        scratch_shapes=[
            pltpu.VMEM((_B, _C), jnp.float32),
            pltpu.VMEM((_B, _C), jnp.int32),
        ],
    )(logits)
    return idx.reshape(_B)
